# baseline (speedup 1.0000x reference)
import functools

import jax
import jax.numpy as jnp
from jax import lax
from jax.experimental import pallas as pl
from jax.experimental.pallas import tpu as pltpu
from jax.experimental.pallas import tpu_sc as plsc

B, N, C = 4, 8192, 16
S, K = 1024, 32
R2 = 0.2 * 0.2
F = 32
NW = 32
ROWS_W = (K * B * S) // NW
CH = 128
NCH = ROWS_W // CH
INNER = 8


def _fps_body(xt_ref, xo_ref, yo_ref, zo_ref):
    X = xt_ref[0]
    Y = xt_ref[1]
    Z = xt_ref[2]
    ir = lax.broadcasted_iota(jnp.int32, (B, 64, 128), 1)
    ic = lax.broadcasted_iota(jnp.int32, (B, 64, 128), 2)
    iota3 = ir * 128 + ic

    def step(t, carry):
        dist, far = carry
        sel = iota3 == far[:, None, None]
        cx = jnp.sum(jnp.where(sel, X, 0.0), axis=(1, 2))
        cy = jnp.sum(jnp.where(sel, Y, 0.0), axis=(1, 2))
        cz = jnp.sum(jnp.where(sel, Z, 0.0), axis=(1, 2))
        xo_ref[pl.ds(t, 1), :] = cx[None, :]
        yo_ref[pl.ds(t, 1), :] = cy[None, :]
        zo_ref[pl.ds(t, 1), :] = cz[None, :]
        dx = X - cx[:, None, None]
        dy = Y - cy[:, None, None]
        dz = Z - cz[:, None, None]
        d = (dx * dx + dy * dy) + (dz * dz)
        dist = jnp.minimum(dist, d)
        m = jnp.max(dist, axis=(1, 2))
        cand = jnp.where(dist == m[:, None, None], iota3, N)
        far = jnp.min(cand, axis=(1, 2))
        return dist, far

    dist0 = jnp.full((B, 64, 128), 1e10, jnp.float32)
    far0 = jnp.zeros((B,), jnp.int32)
    lax.fori_loop(0, S, step, (dist0, far0))


def _fps(xt4):
    shp = jax.ShapeDtypeStruct((S, B), jnp.float32)
    return pl.pallas_call(
        _fps_body,
        out_shape=(shp, shp, shp),
    )(xt4)


T_BQ = 256


def _ballq_body(nx_ref, xt_ref, out_ref):
    aT = nx_ref[0]
    xb = xt_ref[0]
    na = jnp.sum(aT * aT, axis=0, keepdims=True)
    Xc = xb[:, 0:1]
    Yc = xb[:, 1:2]
    Zc = xb[:, 2:3]
    nb = (Xc * Xc + Yc * Yc) + (Zc * Zc)
    xr = xb.astype(jnp.bfloat16).astype(jnp.float32)
    ar = aT.astype(jnp.bfloat16).astype(jnp.float32)
    px = jnp.broadcast_to(xr[:, 0:1], (N, T_BQ)) * ar[0:1, :]
    py = jnp.broadcast_to(xr[:, 1:2], (N, T_BQ)) * ar[1:2, :]
    pz = jnp.broadcast_to(xr[:, 2:3], (N, T_BQ)) * ar[2:3, :]
    ab = (px + py) + pz
    sqr = (na - 2.0 * ab) + nb
    iota_s = lax.broadcasted_iota(jnp.int32, (N, T_BQ), 0)
    v = jnp.where(sqr > R2, N, iota_s)
    for k in range(K):
        cur = jnp.min(v, axis=0, keepdims=True)
        out_ref[0, k : k + 1, :] = cur
        v = jnp.where(v == cur, N, v)
    ii = out_ref[0]
    first = ii[0:1, :]
    ii = jnp.where(ii == N, first, ii)
    out_ref[0] = jnp.minimum(ii, N - 1)


def _ballq(nxT, xyzp):
    return pl.pallas_call(
        _ballq_body,
        grid=(B, S // T_BQ),
        in_specs=[
            pl.BlockSpec((1, 8, T_BQ), lambda b, s: (b, 0, s)),
            pl.BlockSpec((1, N, 8), lambda b, s: (b, 0, 0)),
        ],
        out_specs=pl.BlockSpec((1, K, T_BQ), lambda b, s: (b, 0, s)),
        out_shape=jax.ShapeDtypeStruct((B, K, S), jnp.int32),
    )(nxT, xyzp)


def _sc_gather(feat, gidx3):
    mesh = plsc.VectorSubcoreMesh(core_axis_name="c", subcore_axis_name="s")

    @functools.partial(
        pl.kernel,
        mesh=mesh,
        compiler_params=pltpu.CompilerParams(use_tc_tiling_on_sc=False),
        out_type=jax.ShapeDtypeStruct((K * B * S, F), jnp.float32),
        scratch_types=[
            pltpu.VMEM((NCH, CH), jnp.int32),
            pltpu.VMEM((INNER * CH, F), jnp.float32),
            pltpu.SemaphoreType.DMA,
        ],
    )
    def gath(feat_hbm, gidx_hbm, out_hbm, idx_v, rows_v, sem):
        wid = lax.axis_index("s") * 2 + lax.axis_index("c")
        pltpu.sync_copy(gidx_hbm.at[wid], idx_v)

        def outer(jo, _):
            cps = []
            for jj in range(INNER):
                cp = pltpu.async_copy(
                    feat_hbm.at[idx_v.at[jo * INNER + jj]],
                    rows_v.at[pl.ds(jj * CH, CH)],
                    sem,
                )
                cps.append(cp)
            for cp in cps:
                cp.wait()
            pltpu.sync_copy(
                rows_v,
                out_hbm.at[pl.ds(wid * ROWS_W + jo * (INNER * CH), INNER * CH)],
            )
            return 0

        lax.fori_loop(0, NCH // INNER, outer, 0)

    return gath(feat, gidx3)


T_MLP = 256


def _mlp_body(g_ref, nx_ref, w0_ref, b0_ref, w1_ref, b1_ref, w2_ref, b2_ref,
              out_ref):
    nx = nx_ref[...]
    c = jnp.dot(nx, w0_ref[pl.ds(0, 8), :],
                preferred_element_type=jnp.float32)
    bias0 = b0_ref[...] - c
    w0 = w0_ref[...]
    w1 = w1_ref[...]
    w2 = w2_ref[...]
    b1 = b1_ref[...]
    b2 = b2_ref[...]
    acc = jnp.zeros((T_MLP, 64), jnp.float32)
    for k in range(K):
        g = g_ref[k]
        h = jnp.maximum(
            jnp.dot(g, w0, preferred_element_type=jnp.float32) + bias0, 0.0)
        h = jnp.maximum(
            jnp.dot(h, w1, preferred_element_type=jnp.float32) + b1, 0.0)
        h = jnp.maximum(
            jnp.dot(h, w2, preferred_element_type=jnp.float32) + b2, 0.0)
        acc = jnp.maximum(acc, h)
    out_ref[...] = acc


def _mlp(gk, nxf, w0p, b0, w1, b1, w2, b2):
    BS = B * S
    return pl.pallas_call(
        _mlp_body,
        grid=(BS // T_MLP,),
        in_specs=[
            pl.BlockSpec((K, T_MLP, F), lambda i: (0, i, 0)),
            pl.BlockSpec((T_MLP, 8), lambda i: (i, 0)),
            pl.BlockSpec((F, 32), lambda i: (0, 0)),
            pl.BlockSpec((1, 32), lambda i: (0, 0)),
            pl.BlockSpec((32, 32), lambda i: (0, 0)),
            pl.BlockSpec((1, 32), lambda i: (0, 0)),
            pl.BlockSpec((32, 64), lambda i: (0, 0)),
            pl.BlockSpec((1, 64), lambda i: (0, 0)),
        ],
        out_specs=pl.BlockSpec((T_MLP, 64), lambda i: (i, 0)),
        out_shape=jax.ShapeDtypeStruct((BS, 64), jnp.float32),
    )(gk, nxf, w0p, b0, w1, b1, w2, b2)


def kernel(xyz, points, W0, b0, W1, b1, W2, b2):
    xt = jnp.transpose(xyz, (2, 0, 1))
    xo, yo, zo = _fps(xt.reshape(3, B, 64, 128))
    new_xyz = jnp.stack([xo.T, yo.T, zo.T], axis=-1)

    xyzp = jnp.concatenate(
        [xyz, jnp.zeros((B, N, 5), jnp.float32)], axis=-1)
    nxT = jnp.stack([xo.T, yo.T, zo.T], axis=1)
    nxT = jnp.concatenate(
        [nxT, jnp.zeros((B, 5, S), jnp.float32)], axis=1)
    idxT = _ballq(nxT, xyzp)
    idx = jnp.transpose(idxT, (0, 2, 1))

    feat = jnp.concatenate(
        [xyz, points, jnp.zeros((B, N, F - 3 - C), jnp.float32)],
        axis=-1).reshape(B * N, F)
    gidx = (idx + (jnp.arange(B, dtype=jnp.int32) * N)[:, None, None])
    gidx3 = jnp.transpose(gidx, (2, 0, 1)).reshape(NW, NCH, CH)
    grouped = _sc_gather(feat, gidx3)

    w0p = jnp.concatenate(
        [W0, jnp.zeros((F - (3 + C), W0.shape[1]), jnp.float32)], axis=0)
    new_points = _mlp(
        grouped.reshape(K, B * S, F),
        jnp.concatenate([new_xyz, jnp.zeros((B, S, 5), jnp.float32)],
                        axis=-1).reshape(B * S, 8),
        w0p, b0[None, :], W1, b1[None, :], W2, b2[None, :])
    return new_xyz, new_points.reshape(B, S, 64), idx


# speedup vs baseline: 1.1146x; 1.1146x over previous
import functools

import jax
import jax.numpy as jnp
from jax import lax
from jax.experimental import pallas as pl
from jax.experimental.pallas import tpu as pltpu
from jax.experimental.pallas import tpu_sc as plsc

B, N, C = 4, 8192, 16
S, K = 1024, 32
R2 = 0.2 * 0.2
F = 32
NW = 32
ROWS_W = (K * B * S) // NW
CH = 128
NCH = ROWS_W // CH
INNER = 8


def _fps_body(xt_ref, xo_ref, yo_ref, zo_ref):
    X = xt_ref[0]
    Y = xt_ref[1]
    Z = xt_ref[2]
    ir = lax.broadcasted_iota(jnp.int32, (B, 64, 128), 1)
    ic = lax.broadcasted_iota(jnp.int32, (B, 64, 128), 2)
    iota3 = ir * 128 + ic

    def step(t, carry):
        dist, far = carry
        sel = iota3 == far[:, None, None]
        cx = jnp.sum(jnp.where(sel, X, 0.0), axis=(1, 2))
        cy = jnp.sum(jnp.where(sel, Y, 0.0), axis=(1, 2))
        cz = jnp.sum(jnp.where(sel, Z, 0.0), axis=(1, 2))
        xo_ref[pl.ds(t, 1), :] = cx[None, :]
        yo_ref[pl.ds(t, 1), :] = cy[None, :]
        zo_ref[pl.ds(t, 1), :] = cz[None, :]
        dx = X - cx[:, None, None]
        dy = Y - cy[:, None, None]
        dz = Z - cz[:, None, None]
        d = (dx * dx + dy * dy) + (dz * dz)
        dist = jnp.minimum(dist, d)
        m = jnp.max(dist, axis=(1, 2))
        cand = jnp.where(dist == m[:, None, None], iota3, N)
        far = jnp.min(cand, axis=(1, 2))
        return dist, far

    dist0 = jnp.full((B, 64, 128), 1e10, jnp.float32)
    far0 = jnp.zeros((B,), jnp.int32)
    lax.fori_loop(0, S, step, (dist0, far0))


def _fps(xt4):
    shp = jax.ShapeDtypeStruct((S, B), jnp.float32)
    return pl.pallas_call(
        _fps_body,
        out_shape=(shp, shp, shp),
    )(xt4)


T_BQ = 512


def _ballq_body(nx_ref, xt_ref, out_ref):
    aT = nx_ref[0]
    xb = xt_ref[0]
    na = jnp.sum(aT * aT, axis=0, keepdims=True)
    Xc = xb[:, 0:1]
    Yc = xb[:, 1:2]
    Zc = xb[:, 2:3]
    nb = (Xc * Xc + Yc * Yc) + (Zc * Zc)
    ab = jnp.dot(xb.astype(jnp.bfloat16), aT.astype(jnp.bfloat16),
                 preferred_element_type=jnp.float32)
    sqr = (na - 2.0 * ab) + nb
    iota_s = lax.broadcasted_iota(jnp.int32, (N, T_BQ), 0)
    v = jnp.where(sqr > R2, N, iota_s)
    for k in range(K):
        cur = jnp.min(v, axis=0, keepdims=True)
        out_ref[0, k : k + 1, :] = cur
        v = jnp.where(v == cur, N, v)
    ii = out_ref[0]
    first = ii[0:1, :]
    ii = jnp.where(ii == N, first, ii)
    out_ref[0] = jnp.minimum(ii, N - 1)


def _ballq(nxT, xyzp):
    return pl.pallas_call(
        _ballq_body,
        grid=(B, S // T_BQ),
        in_specs=[
            pl.BlockSpec((1, 8, T_BQ), lambda b, s: (b, 0, s)),
            pl.BlockSpec((1, N, 8), lambda b, s: (b, 0, 0)),
        ],
        out_specs=pl.BlockSpec((1, K, T_BQ), lambda b, s: (b, 0, s)),
        out_shape=jax.ShapeDtypeStruct((B, K, S), jnp.int32),
    )(nxT, xyzp)


def _sc_gather(feat, gidx3):
    mesh = plsc.VectorSubcoreMesh(core_axis_name="c", subcore_axis_name="s")

    @functools.partial(
        pl.kernel,
        mesh=mesh,
        compiler_params=pltpu.CompilerParams(use_tc_tiling_on_sc=False),
        out_type=jax.ShapeDtypeStruct((K * B * S, F), jnp.float32),
        scratch_types=[
            pltpu.VMEM((NCH, CH), jnp.int32),
            pltpu.VMEM((INNER * CH, F), jnp.float32),
            pltpu.SemaphoreType.DMA,
        ],
    )
    def gath(feat_hbm, gidx_hbm, out_hbm, idx_v, rows_v, sem):
        wid = lax.axis_index("s") * 2 + lax.axis_index("c")
        pltpu.sync_copy(gidx_hbm.at[wid], idx_v)

        def outer(jo, _):
            cps = []
            for jj in range(INNER):
                cp = pltpu.async_copy(
                    feat_hbm.at[idx_v.at[jo * INNER + jj]],
                    rows_v.at[pl.ds(jj * CH, CH)],
                    sem,
                )
                cps.append(cp)
            for cp in cps:
                cp.wait()
            pltpu.sync_copy(
                rows_v,
                out_hbm.at[pl.ds(wid * ROWS_W + jo * (INNER * CH), INNER * CH)],
            )
            return 0

        lax.fori_loop(0, NCH // INNER, outer, 0)

    return gath(feat, gidx3)


T_MLP = 256


def _mlp_body(g_ref, nx_ref, w0_ref, b0_ref, w1_ref, b1_ref, w2_ref, b2_ref,
              out_ref):
    nx = nx_ref[...]
    c = jnp.dot(nx, w0_ref[pl.ds(0, 8), :],
                preferred_element_type=jnp.float32)
    bias0 = b0_ref[...] - c
    w0 = w0_ref[...]
    w1 = w1_ref[...]
    w2 = w2_ref[...]
    b1 = b1_ref[...]
    b2 = b2_ref[...]
    acc = jnp.zeros((T_MLP, 64), jnp.float32)
    for k in range(K):
        g = g_ref[k]
        h = jnp.maximum(
            jnp.dot(g, w0, preferred_element_type=jnp.float32) + bias0, 0.0)
        h = jnp.maximum(
            jnp.dot(h, w1, preferred_element_type=jnp.float32) + b1, 0.0)
        h = jnp.maximum(
            jnp.dot(h, w2, preferred_element_type=jnp.float32) + b2, 0.0)
        acc = jnp.maximum(acc, h)
    out_ref[...] = acc


def _mlp(gk, nxf, w0p, b0, w1, b1, w2, b2):
    BS = B * S
    return pl.pallas_call(
        _mlp_body,
        grid=(BS // T_MLP,),
        in_specs=[
            pl.BlockSpec((K, T_MLP, F), lambda i: (0, i, 0)),
            pl.BlockSpec((T_MLP, 8), lambda i: (i, 0)),
            pl.BlockSpec((F, 32), lambda i: (0, 0)),
            pl.BlockSpec((1, 32), lambda i: (0, 0)),
            pl.BlockSpec((32, 32), lambda i: (0, 0)),
            pl.BlockSpec((1, 32), lambda i: (0, 0)),
            pl.BlockSpec((32, 64), lambda i: (0, 0)),
            pl.BlockSpec((1, 64), lambda i: (0, 0)),
        ],
        out_specs=pl.BlockSpec((T_MLP, 64), lambda i: (i, 0)),
        out_shape=jax.ShapeDtypeStruct((BS, 64), jnp.float32),
    )(gk, nxf, w0p, b0, w1, b1, w2, b2)


def kernel(xyz, points, W0, b0, W1, b1, W2, b2):
    xt = jnp.transpose(xyz, (2, 0, 1))
    xo, yo, zo = _fps(xt.reshape(3, B, 64, 128))
    new_xyz = jnp.stack([xo.T, yo.T, zo.T], axis=-1)

    xyzp = jnp.concatenate(
        [xyz, jnp.zeros((B, N, 5), jnp.float32)], axis=-1)
    nxT = jnp.stack([xo.T, yo.T, zo.T], axis=1)
    nxT = jnp.concatenate(
        [nxT, jnp.zeros((B, 5, S), jnp.float32)], axis=1)
    idxT = _ballq(nxT, xyzp)
    idx = jnp.transpose(idxT, (0, 2, 1))

    feat = jnp.concatenate(
        [xyz, points, jnp.zeros((B, N, F - 3 - C), jnp.float32)],
        axis=-1).reshape(B * N, F)
    gidx = (idx + (jnp.arange(B, dtype=jnp.int32) * N)[:, None, None])
    gidx3 = jnp.transpose(gidx, (2, 0, 1)).reshape(NW, NCH, CH)
    grouped = _sc_gather(feat, gidx3)

    w0p = jnp.concatenate(
        [W0, jnp.zeros((F - (3 + C), W0.shape[1]), jnp.float32)], axis=0)
    new_points = _mlp(
        grouped.reshape(K, B * S, F),
        jnp.concatenate([new_xyz, jnp.zeros((B, S, 5), jnp.float32)],
                        axis=-1).reshape(B * S, 8),
        w0p, b0[None, :], W1, b1[None, :], W2, b2[None, :])
    return new_xyz, new_points.reshape(B, S, 64), idx


# speedup vs baseline: 1.1375x; 1.0205x over previous
import functools

import jax
import jax.numpy as jnp
from jax import lax
from jax.experimental import pallas as pl
from jax.experimental.pallas import tpu as pltpu
from jax.experimental.pallas import tpu_sc as plsc

B, N, C = 4, 8192, 16
S, K = 1024, 32
R2 = 0.2 * 0.2
F = 32
NW = 32
ROWS_W = (K * B * S) // NW
CH = 128
NCH = ROWS_W // CH
INNER = 8


def _fps_body(xt_ref, xo_ref, yo_ref, zo_ref):
    X = xt_ref[0]
    Y = xt_ref[1]
    Z = xt_ref[2]
    ir = lax.broadcasted_iota(jnp.int32, (B, 64, 128), 1)
    ic = lax.broadcasted_iota(jnp.int32, (B, 64, 128), 2)
    iota3 = ir * 128 + ic

    def step(t, carry):
        dist, far = carry
        sel = iota3 == far[:, None, None]
        cx = jnp.sum(jnp.where(sel, X, 0.0), axis=(1, 2))
        cy = jnp.sum(jnp.where(sel, Y, 0.0), axis=(1, 2))
        cz = jnp.sum(jnp.where(sel, Z, 0.0), axis=(1, 2))
        xo_ref[pl.ds(t, 1), :] = cx[None, :]
        yo_ref[pl.ds(t, 1), :] = cy[None, :]
        zo_ref[pl.ds(t, 1), :] = cz[None, :]
        dx = X - cx[:, None, None]
        dy = Y - cy[:, None, None]
        dz = Z - cz[:, None, None]
        d = (dx * dx + dy * dy) + (dz * dz)
        dist = jnp.minimum(dist, d)
        m = jnp.max(dist, axis=(1, 2))
        cand = jnp.where(dist == m[:, None, None], iota3, N)
        far = jnp.min(cand, axis=(1, 2))
        return dist, far

    dist0 = jnp.full((B, 64, 128), 1e10, jnp.float32)
    far0 = jnp.zeros((B,), jnp.int32)
    lax.fori_loop(0, S, step, (dist0, far0))


def _fps(xt4):
    shp = jax.ShapeDtypeStruct((S, B), jnp.float32)
    return pl.pallas_call(
        _fps_body,
        out_shape=(shp, shp, shp),
    )(xt4)


T_BQ = 512


def _ballq_body(nx_ref, xt_ref, out_ref):
    aT = nx_ref[0]
    xb = xt_ref[0]
    na = jnp.sum(aT * aT, axis=0, keepdims=True)
    Xc = xb[:, 0:1]
    Yc = xb[:, 1:2]
    Zc = xb[:, 2:3]
    nb = (Xc * Xc + Yc * Yc) + (Zc * Zc)
    ab = jnp.dot(xb.astype(jnp.bfloat16), aT.astype(jnp.bfloat16),
                 preferred_element_type=jnp.float32)
    sqr = (na - 2.0 * ab) + nb
    iota_s = lax.broadcasted_iota(jnp.int32, (N, T_BQ), 0)
    v = jnp.where(sqr > R2, N, iota_s)
    for k in range(K):
        cur = jnp.min(v, axis=0, keepdims=True)
        out_ref[0, k : k + 1, :] = cur
        v = jnp.where(v == cur, N, v)
    ii = out_ref[0]
    first = ii[0:1, :]
    ii = jnp.where(ii == N, first, ii)
    out_ref[0] = jnp.minimum(ii, N - 1)


def _ballq(nxT, xyzp):
    return pl.pallas_call(
        _ballq_body,
        grid=(B, S // T_BQ),
        in_specs=[
            pl.BlockSpec((1, 8, T_BQ), lambda b, s: (b, 0, s)),
            pl.BlockSpec((1, N, 8), lambda b, s: (b, 0, 0)),
        ],
        out_specs=pl.BlockSpec((1, K, T_BQ), lambda b, s: (b, 0, s)),
        out_shape=jax.ShapeDtypeStruct((B, K, S), jnp.int32),
    )(nxT, xyzp)


def _sc_gather(feat, gidx3):
    mesh = plsc.VectorSubcoreMesh(core_axis_name="c", subcore_axis_name="s")

    @functools.partial(
        pl.kernel,
        mesh=mesh,
        compiler_params=pltpu.CompilerParams(use_tc_tiling_on_sc=False),
        out_type=jax.ShapeDtypeStruct((K * B * S, F), jnp.float32),
        scratch_types=[
            pltpu.VMEM((NCH, CH), jnp.int32),
            pltpu.VMEM((INNER * CH, F), jnp.float32),
            pltpu.SemaphoreType.DMA,
        ],
    )
    def gath(feat_hbm, gidx_hbm, out_hbm, idx_v, rows_v, sem):
        wid = lax.axis_index("s") * 2 + lax.axis_index("c")
        pltpu.sync_copy(gidx_hbm.at[wid], idx_v)

        def outer(jo, _):
            cps = []
            for jj in range(INNER):
                cp = pltpu.async_copy(
                    feat_hbm.at[idx_v.at[jo * INNER + jj]],
                    rows_v.at[pl.ds(jj * CH, CH)],
                    sem,
                )
                cps.append(cp)
            for cp in cps:
                cp.wait()
            pltpu.sync_copy(
                rows_v,
                out_hbm.at[pl.ds(wid * ROWS_W + jo * (INNER * CH), INNER * CH)],
            )
            return 0

        lax.fori_loop(0, NCH // INNER, outer, 0)

    return gath(feat, gidx3)


T_MLP = 512


def _mlp_body(g_ref, nx_ref, w0_ref, b0_ref, w1_ref, b1_ref, w2_ref, b2_ref,
              out_ref):
    nx = nx_ref[...]
    c = jnp.dot(nx, w0_ref[pl.ds(0, 8), :],
                preferred_element_type=jnp.float32)
    bias0 = b0_ref[...] - c
    w0 = w0_ref[...]
    w1 = w1_ref[...]
    w2 = w2_ref[...]
    b1 = b1_ref[...]
    b2 = b2_ref[...]
    acc = jnp.zeros((T_MLP, 64), jnp.float32)
    for k in range(K):
        g = g_ref[k]
        h = jnp.maximum(
            jnp.dot(g, w0, preferred_element_type=jnp.float32) + bias0, 0.0)
        h = jnp.maximum(
            jnp.dot(h, w1, preferred_element_type=jnp.float32) + b1, 0.0)
        h = jnp.maximum(
            jnp.dot(h, w2, preferred_element_type=jnp.float32) + b2, 0.0)
        acc = jnp.maximum(acc, h)
    out_ref[...] = acc


def _mlp(gk, nxf, w0p, b0, w1, b1, w2, b2):
    BS = B * S
    return pl.pallas_call(
        _mlp_body,
        grid=(BS // T_MLP,),
        in_specs=[
            pl.BlockSpec((K, T_MLP, F), lambda i: (0, i, 0)),
            pl.BlockSpec((T_MLP, 8), lambda i: (i, 0)),
            pl.BlockSpec((F, 32), lambda i: (0, 0)),
            pl.BlockSpec((1, 32), lambda i: (0, 0)),
            pl.BlockSpec((32, 32), lambda i: (0, 0)),
            pl.BlockSpec((1, 32), lambda i: (0, 0)),
            pl.BlockSpec((32, 64), lambda i: (0, 0)),
            pl.BlockSpec((1, 64), lambda i: (0, 0)),
        ],
        out_specs=pl.BlockSpec((T_MLP, 64), lambda i: (i, 0)),
        out_shape=jax.ShapeDtypeStruct((BS, 64), jnp.float32),
    )(gk, nxf, w0p, b0, w1, b1, w2, b2)


def kernel(xyz, points, W0, b0, W1, b1, W2, b2):
    xt = jnp.transpose(xyz, (2, 0, 1))
    xo, yo, zo = _fps(xt.reshape(3, B, 64, 128))
    new_xyz = jnp.stack([xo.T, yo.T, zo.T], axis=-1)

    xyzp = jnp.concatenate(
        [xyz, jnp.zeros((B, N, 5), jnp.float32)], axis=-1)
    nxT = jnp.stack([xo.T, yo.T, zo.T], axis=1)
    nxT = jnp.concatenate(
        [nxT, jnp.zeros((B, 5, S), jnp.float32)], axis=1)
    idxT = _ballq(nxT, xyzp)
    idx = jnp.transpose(idxT, (0, 2, 1))

    feat = jnp.concatenate(
        [xyz, points, jnp.zeros((B, N, F - 3 - C), jnp.float32)],
        axis=-1).reshape(B * N, F)
    gidx = (idx + (jnp.arange(B, dtype=jnp.int32) * N)[:, None, None])
    gidx3 = jnp.transpose(gidx, (2, 0, 1)).reshape(NW, NCH, CH)
    grouped = _sc_gather(feat, gidx3)

    w0p = jnp.concatenate(
        [W0, jnp.zeros((F - (3 + C), W0.shape[1]), jnp.float32)], axis=0)
    new_points = _mlp(
        grouped.reshape(K, B * S, F),
        jnp.concatenate([new_xyz, jnp.zeros((B, S, 5), jnp.float32)],
                        axis=-1).reshape(B * S, 8),
        w0p, b0[None, :], W1, b1[None, :], W2, b2[None, :])
    return new_xyz, new_points.reshape(B, S, 64), idx
